# Initial kernel scaffold; baseline (speedup 1.0000x reference)
import functools

import jax
import jax.numpy as jnp
from jax import lax
from jax.experimental import pallas as pl
from jax.experimental.pallas import tpu as pltpu
from jax.experimental.pallas import tpu_sc as plsc

_N = 10000
_E = 320000
_D = 128
_H = 4
_CH = 32

_NC = 2
_NS = 16
_NW = _NC * _NS
_C = 80
_EPW = _E // _NW
_RPT = _N // _NS

_f32 = jnp.float32


_sc_mesh = plsc.VectorSubcoreMesh(core_axis_name="c", subcore_axis_name="s")


@functools.partial(
    pl.kernel,
    out_type=(jax.ShapeDtypeStruct((_E, _D), _f32),
              jax.ShapeDtypeStruct((_E, _D), _f32)),
    mesh=_sc_mesh,
    scratch_types=[
        pltpu.VMEM((_C,), jnp.int32),
        pltpu.VMEM((_C,), jnp.int32),
        pltpu.VMEM((_C, _D), _f32),
        pltpu.VMEM((_C, _D), _f32),
        pltpu.SemaphoreType.DMA,
        pltpu.SemaphoreType.DMA,
    ],
)
def _sc_gather(xl_hbm, xr_hbm, dst_hbm, src_hbm, g1_hbm, g2_hbm,
               idx1, idx2, rows1, rows2, sem1, sem2):
    wid = lax.axis_index("s") * _NC + lax.axis_index("c")
    base0 = wid * _EPW

    def body(j, carry):
        base = base0 + j * _C
        pltpu.sync_copy(dst_hbm.at[pl.ds(base, _C)], idx1)
        pltpu.sync_copy(src_hbm.at[pl.ds(base, _C)], idx2)
        cp1 = pltpu.async_copy(xl_hbm.at[idx1], rows1, sem1)
        cp2 = pltpu.async_copy(xr_hbm.at[idx2], rows2, sem2)
        cp1.wait()
        pltpu.sync_copy(rows1, g1_hbm.at[pl.ds(base, _C)])
        cp2.wait()
        pltpu.sync_copy(rows2, g2_hbm.at[pl.ds(base, _C)])
        return carry

    lax.fori_loop(0, _EPW // _C, body, 0)


@functools.partial(
    pl.kernel,
    out_type=(jax.ShapeDtypeStruct((_NC, _N, _D), _f32),
              jax.ShapeDtypeStruct((_NC, _N, 16), _f32)),
    mesh=_sc_mesh,
    scratch_types=[
        pltpu.VMEM((_C,), jnp.int32),
        pltpu.VMEM((_C, _D), _f32),
        pltpu.VMEM((_C, 16), _f32),
        pltpu.MemorySpace.VMEM_SHARED((_N, _D), _f32),
        pltpu.MemorySpace.VMEM_SHARED((_N, 16), _f32),
    ],
)
def _sc_scatter(u_hbm, ex_hbm, dst_hbm, z128_hbm, z16_hbm, upart_hbm, dpart_hbm,
                idx, ubuf, exbuf, uacc, dacc):
    cid = lax.axis_index("c")
    sid = lax.axis_index("s")
    wid = sid * _NC + cid
    base0 = wid * _EPW

    rb = sid * _RPT
    pltpu.sync_copy(z128_hbm.at[pl.ds(rb, _RPT)], uacc.at[pl.ds(rb, _RPT)])
    pltpu.sync_copy(z16_hbm.at[pl.ds(rb, _RPT)], dacc.at[pl.ds(rb, _RPT)])
    plsc.subcore_barrier()

    def body(j, carry):
        base = base0 + j * _C
        pltpu.sync_copy(dst_hbm.at[pl.ds(base, _C)], idx)
        pltpu.sync_copy(u_hbm.at[pl.ds(base, _C)], ubuf)
        pltpu.sync_copy(ex_hbm.at[pl.ds(base, _C)], exbuf)
        pltpu.sync_copy(ubuf, uacc.at[idx], add=True)
        pltpu.sync_copy(exbuf, dacc.at[idx], add=True)
        return carry

    lax.fori_loop(0, _EPW // _C, body, 0)
    plsc.subcore_barrier()

    pltpu.sync_copy(uacc.at[pl.ds(rb, _RPT)], upart_hbm.at[cid, pl.ds(rb, _RPT)])
    pltpu.sync_copy(dacc.at[pl.ds(rb, _RPT)], dpart_hbm.at[cid, pl.ds(rb, _RPT)])


_BM = 2000
_BE = 2000


def _full(shape):
    return pl.BlockSpec(shape, lambda i: tuple(0 for _ in shape))


def _tc_proj_body(x, wp, bp, h_out):
    h_out[...] = jnp.dot(x[...], wp[...], preferred_element_type=_f32) + bp[...]


def _tc_proj(x, wp, bp):
    return pl.pallas_call(
        _tc_proj_body,
        grid=(_N // _BM,),
        in_specs=[pl.BlockSpec((_BM, _D), lambda i: (i, 0)),
                  _full((_D, _D)), _full((1, _D))],
        out_specs=pl.BlockSpec((_BM, _D), lambda i: (i, 0)),
        out_shape=jax.ShapeDtypeStruct((_N, _D), _f32),
    )(x, wp, bp)


def _tc_lr_body(h, wl, wr, xl_out, xr_out):
    hh = h[...]
    xl_out[...] = jnp.dot(hh, wl[...], preferred_element_type=_f32)
    xr_out[...] = jnp.dot(hh, wr[...], preferred_element_type=_f32)


def _tc_lr(h, wl, wr):
    return pl.pallas_call(
        _tc_lr_body,
        grid=(_N // _BM,),
        in_specs=[pl.BlockSpec((_BM, _D), lambda i: (i, 0)),
                  _full((_D, _D)), _full((_D, _D))],
        out_specs=[pl.BlockSpec((_BM, _D), lambda i: (i, 0)),
                   pl.BlockSpec((_BM, _D), lambda i: (i, 0))],
        out_shape=[jax.ShapeDtypeStruct((_N, _D), _f32),
                   jax.ShapeDtypeStruct((_N, _D), _f32)],
    )(h, wl, wr)


def _tc_edge_body(g1, g2, amap, bexp, u_out, ex_out):
    s = g1[...] + g2[...]
    s = jnp.where(s > 0, s, 0.2 * s)
    e = jnp.exp(jnp.dot(s, amap[...], preferred_element_type=_f32))
    ex_out[...] = e
    u_out[...] = g2[...] * jnp.dot(e, bexp[...], preferred_element_type=_f32)


def _tc_edge(g1, g2, amap, bexp):
    return pl.pallas_call(
        _tc_edge_body,
        grid=(_E // _BE,),
        in_specs=[pl.BlockSpec((_BE, _D), lambda i: (i, 0)),
                  pl.BlockSpec((_BE, _D), lambda i: (i, 0)),
                  _full((_D, 16)), _full((16, _D))],
        out_specs=[pl.BlockSpec((_BE, _D), lambda i: (i, 0)),
                   pl.BlockSpec((_BE, 16), lambda i: (i, 0))],
        out_shape=[jax.ShapeDtypeStruct((_E, _D), _f32),
                   jax.ShapeDtypeStruct((_E, 16), _f32)],
    )(g1, g2, amap, bexp)


def _tc_combine_body(up, dp, hprev, bias, ng, nb, bexp, h_out):
    u = up[0] + up[1]
    den = dp[0] + dp[1]
    denx = jnp.dot(den, bexp[...], preferred_element_type=_f32) + 1e-16
    y = hprev[...] + u / denx + bias[...]
    mu = jnp.mean(y, axis=-1, keepdims=True)
    var = jnp.mean((y - mu) ** 2, axis=-1, keepdims=True)
    h_out[...] = (y - mu) / jnp.sqrt(var + 1e-5) * ng[...] + nb[...]


def _tc_combine(upart, dpart, hprev, bias, ng, nb, bexp):
    return pl.pallas_call(
        _tc_combine_body,
        grid=(_N // _BM,),
        in_specs=[pl.BlockSpec((_NC, _BM, _D), lambda i: (0, i, 0)),
                  pl.BlockSpec((_NC, _BM, 16), lambda i: (0, i, 0)),
                  pl.BlockSpec((_BM, _D), lambda i: (i, 0)),
                  _full((1, _D)), _full((1, _D)), _full((1, _D)),
                  _full((16, _D))],
        out_specs=pl.BlockSpec((_BM, _D), lambda i: (i, 0)),
        out_shape=jax.ShapeDtypeStruct((_N, _D), _f32),
    )(upart, dpart, hprev, bias, ng, nb, bexp)


def _gelu(x):
    return 0.5 * x * (1.0 + lax.erf(x * 0.7071067811865476))


def _tc_final_body(h, fng, fnb, pw1, pb1, pw2, pb2, vw1, vb1, vw2, vb2,
                   vol_out, pl_out):
    y = h[...]
    mu = jnp.mean(y, axis=-1, keepdims=True)
    var = jnp.mean((y - mu) ** 2, axis=-1, keepdims=True)
    hn = (y - mu) / jnp.sqrt(var + 1e-5) * fng[...] + fnb[...]
    ph = _gelu(jnp.dot(hn, pw1[...], preferred_element_type=_f32) + pb1[...])
    pl_out[...] = jnp.dot(ph, pw2[...], preferred_element_type=_f32) + pb2[...]
    vh = _gelu(jnp.dot(hn, vw1[...], preferred_element_type=_f32) + vb1[...])
    vol_out[...] = jnp.dot(vh, vw2[...], preferred_element_type=_f32) + vb2[...]


def _tc_final(h, fng, fnb, pw1, pb1, pw2p, pb2p, vw1, vb1, vw2, vb2):
    return pl.pallas_call(
        _tc_final_body,
        grid=(_N // _BM,),
        in_specs=[pl.BlockSpec((_BM, _D), lambda i: (i, 0)),
                  _full((1, _D)), _full((1, _D)),
                  _full((_D, _D // 2)), _full((1, _D // 2)),
                  _full((_D // 2, 8)), _full((1, 8)),
                  _full((_D, _D // 2)), _full((1, _D // 2)),
                  _full((_D // 2, _D)), _full((1, _D))],
        out_specs=[pl.BlockSpec((_BM, _D), lambda i: (i, 0)),
                   pl.BlockSpec((_BM, 8), lambda i: (i, 0))],
        out_shape=[jax.ShapeDtypeStruct((_N, _D), _f32),
                   jax.ShapeDtypeStruct((_N, 8), _f32)],
    )(h, fng, fnb, pw1, pb1, pw2p, pb2p, vw1, vb1, vw2, vb2)


def kernel(x, edge_index, params):
    src = edge_index[0]
    dst = edge_index[1]
    p = params

    head_of = jnp.arange(_D, dtype=jnp.int32) // _CH
    bexp = (head_of[None, :] == jnp.arange(16, dtype=jnp.int32)[:, None]
            ).astype(_f32)
    z128 = jnp.zeros((_N, _D), _f32)
    z16 = jnp.zeros((_N, 16), _f32)
    row1 = lambda v: v.reshape(1, -1)

    h = _tc_proj(x, p['Wp'], row1(p['bp']))
    for lp in p['layers']:
        xl, xr = _tc_lr(h, lp['Wl'], lp['Wr'])
        g1, g2 = _sc_gather(xl, xr, dst, src)
        amap = bexp.T * lp['att'].reshape(_D)[:, None]
        u, ex = _tc_edge(g1, g2, amap, bexp)
        upart, dpart = _sc_scatter(u, ex, dst, z128, z16)
        h = _tc_combine(upart, dpart, h, row1(lp['bias']),
                        row1(lp['ng']), row1(lp['nb']), bexp)

    pw2p = jnp.pad(p['pW2'], ((0, 0), (0, 7)))
    pb2p = jnp.pad(p['pb2'], (0, 7))
    vol, pl8 = _tc_final(h, row1(p['fng']), row1(p['fnb']),
                         p['pW1'], row1(p['pb1']), pw2p, row1(pb2p),
                         p['vW1'], row1(p['vb1']), p['vW2'], row1(p['vb2']))
    return vol, pl8[:, :1]


# speedup vs baseline: 23.4079x; 23.4079x over previous
import functools

import jax
import jax.numpy as jnp
from jax import lax
from jax.experimental import pallas as pl
from jax.experimental.pallas import tpu as pltpu
from jax.experimental.pallas import tpu_sc as plsc

_N = 10000
_E = 320000
_D = 128
_H = 4
_CH = 32

_NC = 2
_NS = 16
_NW = _NC * _NS
_C = 80
_EPW = _E // _NW
_RPT = _N // _NS

_f32 = jnp.float32


@functools.cache
def _sc_gather_call():
    mesh = plsc.VectorSubcoreMesh(core_axis_name="c", subcore_axis_name="s",
                                  num_cores=_NC, num_subcores=_NS)
    return pl.kernel(
        _sc_gather_body,
        out_type=(jax.ShapeDtypeStruct((_E, _D), _f32),
                  jax.ShapeDtypeStruct((_E, _D), _f32)),
        mesh=mesh,
        scratch_types=[
            pltpu.VMEM((_C,), jnp.int32),
            pltpu.VMEM((_C,), jnp.int32),
            pltpu.VMEM((_C, _D), _f32),
            pltpu.VMEM((_C, _D), _f32),
            pltpu.SemaphoreType.DMA,
            pltpu.SemaphoreType.DMA,
        ],
    )


def _sc_gather(xl, xr, dst, src):
    return _sc_gather_call()(xl, xr, dst, src)


def _sc_gather_body(xl_hbm, xr_hbm, dst_hbm, src_hbm, g1_hbm, g2_hbm,
                    idx1, idx2, rows1, rows2, sem1, sem2):
    wid = lax.axis_index("s") * _NC + lax.axis_index("c")
    base0 = wid * _EPW

    def body(j, carry):
        base = base0 + j * _C
        pltpu.sync_copy(dst_hbm.at[pl.ds(base, _C)], idx1)
        pltpu.sync_copy(src_hbm.at[pl.ds(base, _C)], idx2)
        cp1 = pltpu.async_copy(xl_hbm.at[idx1], rows1, sem1)
        cp2 = pltpu.async_copy(xr_hbm.at[idx2], rows2, sem2)
        cp1.wait()
        pltpu.sync_copy(rows1, g1_hbm.at[pl.ds(base, _C)])
        cp2.wait()
        pltpu.sync_copy(rows2, g2_hbm.at[pl.ds(base, _C)])
        return carry

    lax.fori_loop(0, _EPW // _C, body, 0)


@functools.cache
def _sc_scatter_call():
    mesh = plsc.VectorSubcoreMesh(core_axis_name="c", subcore_axis_name="s",
                                  num_cores=_NC, num_subcores=_NS)
    return pl.kernel(
        _sc_scatter_body,
        out_type=(jax.ShapeDtypeStruct((_NC, _N, _D), _f32),
                  jax.ShapeDtypeStruct((_NC, _N, _D), _f32)),
        mesh=mesh,
        scratch_types=[
            pltpu.VMEM((_C,), jnp.int32),
            pltpu.VMEM((_C, _D), _f32),
            pltpu.MemorySpace.VMEM_SHARED((_N, _D), _f32),
        ],
    )


def _sc_scatter(u, exx, dst, z128):
    return _sc_scatter_call()(u, exx, dst, z128)


def _sc_scatter_body(u_hbm, exx_hbm, dst_hbm, z128_hbm, upart_hbm,
                     dpart_hbm, idx, ubuf, uacc):
    cid = lax.axis_index("c")
    sid = lax.axis_index("s")
    wid = sid * _NC + cid
    base0 = wid * _EPW

    rb = sid * 640
    nz = jnp.where(sid < _NS - 1, 8, 5)

    for src_hbm, part_hbm in ((u_hbm, upart_hbm), (exx_hbm, dpart_hbm)):
        pltpu.sync_copy(z128_hbm.at[pl.ds(0, _C)], ubuf)

        def zbody(j, carry):
            pltpu.sync_copy(ubuf, uacc.at[pl.ds(rb + j * _C, _C)])
            return carry

        lax.fori_loop(0, nz, zbody, 0)
        plsc.subcore_barrier()

        def body(j, carry):
            base = base0 + j * _C
            pltpu.sync_copy(dst_hbm.at[pl.ds(base, _C)], idx)
            pltpu.sync_copy(src_hbm.at[pl.ds(base, _C)], ubuf)
            pltpu.sync_copy(ubuf, uacc.at[idx], add=True)
            return carry

        lax.fori_loop(0, _EPW // _C, body, 0)
        plsc.subcore_barrier()

        def obody(j, carry):
            pltpu.sync_copy(uacc.at[pl.ds(rb + j * _C, _C)], ubuf)
            pltpu.sync_copy(ubuf, part_hbm.at[cid, pl.ds(rb + j * _C, _C)])
            return carry

        lax.fori_loop(0, nz, obody, 0)
        plsc.subcore_barrier()


_BM = 2000
_BE = 2000


def _full(shape):
    return pl.BlockSpec(shape, lambda i: tuple(0 for _ in shape))


def _tc_proj_body(x, wp, bp, h_out):
    h_out[...] = jnp.dot(x[...], wp[...], preferred_element_type=_f32) + bp[...]


def _tc_proj(x, wp, bp):
    return pl.pallas_call(
        _tc_proj_body,
        grid=(_N // _BM,),
        in_specs=[pl.BlockSpec((_BM, _D), lambda i: (i, 0)),
                  _full((_D, _D)), _full((1, _D))],
        out_specs=pl.BlockSpec((_BM, _D), lambda i: (i, 0)),
        out_shape=jax.ShapeDtypeStruct((_N, _D), _f32),
    )(x, wp, bp)


def _tc_lr_body(h, wl, wr, xl_out, xr_out):
    hh = h[...]
    xl_out[...] = jnp.dot(hh, wl[...], preferred_element_type=_f32)
    xr_out[...] = jnp.dot(hh, wr[...], preferred_element_type=_f32)


def _tc_lr(h, wl, wr):
    return pl.pallas_call(
        _tc_lr_body,
        grid=(_N // _BM,),
        in_specs=[pl.BlockSpec((_BM, _D), lambda i: (i, 0)),
                  _full((_D, _D)), _full((_D, _D))],
        out_specs=[pl.BlockSpec((_BM, _D), lambda i: (i, 0)),
                   pl.BlockSpec((_BM, _D), lambda i: (i, 0))],
        out_shape=[jax.ShapeDtypeStruct((_N, _D), _f32),
                   jax.ShapeDtypeStruct((_N, _D), _f32)],
    )(h, wl, wr)


def _tc_edge_body(g1, g2, amap, bexp, u_out, exx_out):
    s = g1[...] + g2[...]
    s = jnp.where(s > 0, s, 0.2 * s)
    e = jnp.exp(jnp.dot(s, amap[...], preferred_element_type=_f32))
    exx = jnp.dot(e, bexp[...], preferred_element_type=_f32)
    exx_out[...] = exx
    u_out[...] = g2[...] * exx


def _tc_edge(g1, g2, amap, bexp):
    return pl.pallas_call(
        _tc_edge_body,
        grid=(_E // _BE,),
        in_specs=[pl.BlockSpec((_BE, _D), lambda i: (i, 0)),
                  pl.BlockSpec((_BE, _D), lambda i: (i, 0)),
                  _full((_D, 16)), _full((16, _D))],
        out_specs=[pl.BlockSpec((_BE, _D), lambda i: (i, 0)),
                   pl.BlockSpec((_BE, _D), lambda i: (i, 0))],
        out_shape=[jax.ShapeDtypeStruct((_E, _D), _f32),
                   jax.ShapeDtypeStruct((_E, _D), _f32)],
    )(g1, g2, amap, bexp)


def _tc_combine_body(up, dp, hprev, bias, ng, nb, h_out):
    u = up[0] + up[1]
    denx = dp[0] + dp[1] + 1e-16
    y = hprev[...] + u / denx + bias[...]
    mu = jnp.mean(y, axis=-1, keepdims=True)
    var = jnp.mean((y - mu) ** 2, axis=-1, keepdims=True)
    h_out[...] = (y - mu) / jnp.sqrt(var + 1e-5) * ng[...] + nb[...]


def _tc_combine(upart, dpart, hprev, bias, ng, nb):
    return pl.pallas_call(
        _tc_combine_body,
        grid=(_N // _BM,),
        in_specs=[pl.BlockSpec((_NC, _BM, _D), lambda i: (0, i, 0)),
                  pl.BlockSpec((_NC, _BM, _D), lambda i: (0, i, 0)),
                  pl.BlockSpec((_BM, _D), lambda i: (i, 0)),
                  _full((1, _D)), _full((1, _D)), _full((1, _D))],
        out_specs=pl.BlockSpec((_BM, _D), lambda i: (i, 0)),
        out_shape=jax.ShapeDtypeStruct((_N, _D), _f32),
    )(upart, dpart, hprev, bias, ng, nb)


def _gelu(x):
    return 0.5 * x * (1.0 + lax.erf(x * 0.7071067811865476))


def _tc_final_body(h, fng, fnb, pw1, pb1, pw2, pb2, vw1, vb1, vw2, vb2,
                   vol_out, pl_out):
    y = h[...]
    mu = jnp.mean(y, axis=-1, keepdims=True)
    var = jnp.mean((y - mu) ** 2, axis=-1, keepdims=True)
    hn = (y - mu) / jnp.sqrt(var + 1e-5) * fng[...] + fnb[...]
    ph = _gelu(jnp.dot(hn, pw1[...], preferred_element_type=_f32) + pb1[...])
    pl_out[...] = jnp.dot(ph, pw2[...], preferred_element_type=_f32) + pb2[...]
    vh = _gelu(jnp.dot(hn, vw1[...], preferred_element_type=_f32) + vb1[...])
    vol_out[...] = jnp.dot(vh, vw2[...], preferred_element_type=_f32) + vb2[...]


def _tc_final(h, fng, fnb, pw1, pb1, pw2p, pb2p, vw1, vb1, vw2, vb2):
    return pl.pallas_call(
        _tc_final_body,
        grid=(_N // _BM,),
        in_specs=[pl.BlockSpec((_BM, _D), lambda i: (i, 0)),
                  _full((1, _D)), _full((1, _D)),
                  _full((_D, _D // 2)), _full((1, _D // 2)),
                  _full((_D // 2, 8)), _full((1, 8)),
                  _full((_D, _D // 2)), _full((1, _D // 2)),
                  _full((_D // 2, _D)), _full((1, _D))],
        out_specs=[pl.BlockSpec((_BM, _D), lambda i: (i, 0)),
                   pl.BlockSpec((_BM, 8), lambda i: (i, 0))],
        out_shape=[jax.ShapeDtypeStruct((_N, _D), _f32),
                   jax.ShapeDtypeStruct((_N, 8), _f32)],
    )(h, fng, fnb, pw1, pb1, pw2p, pb2p, vw1, vb1, vw2, vb2)


def kernel(x, edge_index, params):
    src = edge_index[0]
    dst = edge_index[1]
    p = params

    head_of = jnp.arange(_D, dtype=jnp.int32) // _CH
    bexp = (head_of[None, :] == jnp.arange(16, dtype=jnp.int32)[:, None]
            ).astype(_f32)
    z128 = jnp.zeros((_N, _D), _f32)
    row1 = lambda v: v.reshape(1, -1)

    h = _tc_proj(x, p['Wp'], row1(p['bp']))
    for lp in p['layers']:
        xl, xr = _tc_lr(h, lp['Wl'], lp['Wr'])
        g1, g2 = _sc_gather(xl, xr, dst, src)
        amap = bexp.T * lp['att'].reshape(_D)[:, None]
        u, exx = _tc_edge(g1, g2, amap, bexp)
        upart, dpart = _sc_scatter(u, exx, dst, z128)
        h = _tc_combine(upart, dpart, h, row1(lp['bias']),
                        row1(lp['ng']), row1(lp['nb']))

    pw2p = jnp.pad(p['pW2'], ((0, 0), (0, 7)))
    pb2p = jnp.pad(p['pb2'], (0, 7))
    vol, pl8 = _tc_final(h, row1(p['fng']), row1(p['fnb']),
                         p['pW1'], row1(p['pb1']), pw2p, row1(pb2p),
                         p['vW1'], row1(p['vb1']), p['vW2'], row1(p['vb2']))
    return vol, pl8[:, :1]
